# baseline (speedup 1.0000x reference)
import functools

import jax
import jax.numpy as jnp
from jax import lax
from jax.experimental import pallas as pl
from jax.experimental.pallas import tpu as pltpu
from jax.experimental.pallas import tpu_sc as plsc

N = 10000
NS = 16
R2 = 0.01
CF = 64
NW = 32
QPW = 320
NPAD = NW * QPW
NCH = NPAD // 16
CIN = 80
GROWS = NPAD * NS
VROWS = N * NS
BLK = 2048
QBLK = BLK // NS
GRID = GROWS // BLK
GCH = 128
GNC = GROWS // (NW * GCH)

RB = 512
CB = 2560

_SC_PARAMS = pltpu.CompilerParams(needs_layout_passes=False)


NWORD = NPAD // 4
NWIN = NPAD // 64
WBLK = NWORD // (NPAD // CB)
WCBLK = NWIN // (NPAD // CB)


def _sq_mask(pq_ref, pt_ref):
    pb = pq_ref[...]
    pt = pt_ref[...]
    pr2 = pb * pb
    pn_r = pr2[:, 0:1] + pr2[:, 1:2] + pr2[:, 2:3]
    pc2 = pt * pt
    pn_c = pc2[0:1, :] + pc2[1:2, :] + pc2[2:3, :]
    dot = jnp.dot(pb.astype(jnp.bfloat16), pt.astype(jnp.bfloat16),
                  preferred_element_type=jnp.float32)
    sq = (pn_r + pn_c) - 2.0 * dot
    return (sq <= R2).astype(jnp.bfloat16)


NWC = 512


def _mask_body(pq_ref, pt_ref, pk_ref, bw_ref, t_ref, w1_ref,
               w_ref, wc_ref, u_ref, c_ref):
    mb = _sq_mask(pq_ref, pt_ref)
    w_ref[...] = jnp.dot(mb, pk_ref[...], preferred_element_type=jnp.float32)
    wc_ref[...] = jnp.dot(mb, bw_ref[...], preferred_element_type=jnp.float32)

    @pl.when(pl.program_id(1) == 0)
    def _():
        w1 = w1_ref[...]
        u_ref[...] = jnp.dot(t_ref[...], w1, preferred_element_type=jnp.float32)
        c_ref[...] = jnp.dot(pq_ref[...], w1[0:3, :],
                             preferred_element_type=jnp.float32)


def _maskk(Ppad, PT, PK, BW, T, W1pad):
    return pl.pallas_call(
        _mask_body,
        grid=(NPAD // RB, NPAD // CB),
        in_specs=[
            pl.BlockSpec((RB, 3), lambda r, c: (r, 0)),
            pl.BlockSpec((3, CB), lambda r, c: (0, c)),
            pl.BlockSpec((CB, WBLK), lambda r, c: (0, 0)),
            pl.BlockSpec((CB, 128), lambda r, c: (0, 0)),
            pl.BlockSpec((RB, CIN), lambda r, c: (r, 0)),
            pl.BlockSpec((CIN, CF), lambda r, c: (0, 0)),
        ],
        out_specs=[
            pl.BlockSpec((RB, WBLK), lambda r, c: (r, c)),
            pl.BlockSpec((RB, 128), lambda r, c: (r, c)),
            pl.BlockSpec((RB, CF), lambda r, c: (r, 0)),
            pl.BlockSpec((RB, CF), lambda r, c: (r, 0)),
        ],
        out_shape=[
            jax.ShapeDtypeStruct((NPAD, NWORD), jnp.float32),
            jax.ShapeDtypeStruct((NPAD, NWC), jnp.float32),
            jax.ShapeDtypeStruct((NPAD, CF), jnp.float32),
            jax.ShapeDtypeStruct((NPAD, CF), jnp.float32),
        ],
    )(Ppad, PT, PK, BW, T, W1pad)


_REAL_GRPS = [0, 1, 2, 8, 9, 10, 16, 17, 18, 24, 25, 26]


def _bq_body(w_h, wc_h, out_h, row0, row1, cnt0, cnt1, wlist, cbase, buf,
             stage, sem0, sem1):
    wid = lax.axis_index("s") * 2 + lax.axis_index("c")
    lanes = lax.iota(jnp.int32, 16)
    base = wid * QPW
    wlist[...] = jnp.zeros((16,), jnp.int32)
    cbase[...] = jnp.zeros((16,), jnp.int32)
    pltpu.make_async_copy(w_h.at[base], row0, sem0).start()
    pltpu.make_async_copy(wc_h.at[base], cnt0, sem0).start()

    def process(q, rowv, cntv):
        run = jnp.zeros((16,), jnp.int32)
        nf = jnp.zeros((16,), jnp.int32)
        for g in _REAL_GRPS:
            wbase = 40 * (g // 8) + 16 * (g % 8)
            cwi = cntv[pl.ds(g * 16, 16)].astype(jnp.int32)
            cums = plsc.cumsum(cwi)
            cume = run + cums - cwi
            flag = (cwi > 0) & (cume < NS)
            fpos = nf + plsc.cumsum(flag.astype(jnp.int32)) - 1
            wm = flag & (fpos < 16)
            plsc.store_scatter(wlist, [fpos], wbase + lanes, mask=wm)
            plsc.store_scatter(cbase, [fpos], cume, mask=wm)
            nf = nf + plsc.all_reduce_population_count(flag)
            run = run + cums[jnp.zeros((16,), jnp.int32) + 15]
        wl = wlist[...]
        cb = cbase[...]

        for k in range(16):
            kv = nf > k
            w = wl[k]
            wi = rowv[pl.ds(w * 16, 16)].astype(jnp.int32)
            f0 = wi & 1
            f1 = (wi >> 1) & 1
            f2 = (wi >> 2) & 1
            f3 = (wi >> 3) & 1
            cwl = f0 + f1 + f2 + f3
            pexc = plsc.cumsum(cwl) - cwl
            cnd = w * 64 + 4 * lanes
            pos0 = cb[k] + pexc
            plsc.store_scatter(buf, [pos0], cnd,
                               mask=kv & (f0 > 0) & (pos0 < NS))
            pos1 = pos0 + f0
            plsc.store_scatter(buf, [pos1], cnd + 1,
                               mask=kv & (f1 > 0) & (pos1 < NS))
            pos2 = pos1 + f1
            plsc.store_scatter(buf, [pos2], cnd + 2,
                               mask=kv & (f2 > 0) & (pos2 < NS))
            pos3 = pos2 + f2
            plsc.store_scatter(buf, [pos3], cnd + 3,
                               mask=kv & (f3 > 0) & (pos3 < NS))

        vals = buf[...]
        first = jnp.where(run > 0, vals[jnp.zeros((16,), jnp.int32)], N)
        stage[q, :] = jnp.where(lanes < run, vals, first)

    def pair(t, carry):
        q0 = 2 * t
        q1 = 2 * t + 1
        pltpu.make_async_copy(w_h.at[base + q1], row1, sem1).start()
        pltpu.make_async_copy(wc_h.at[base + q1], cnt1, sem1).start()
        pltpu.make_async_copy(w_h.at[base + q0], row0, sem0).wait()
        pltpu.make_async_copy(wc_h.at[base + q0], cnt0, sem0).wait()
        process(q0, row0, cnt0)

        @pl.when(t < QPW // 2 - 1)
        def _():
            pltpu.make_async_copy(w_h.at[base + q1 + 1], row0, sem0).start()
            pltpu.make_async_copy(wc_h.at[base + q1 + 1], cnt0, sem0).start()

        pltpu.make_async_copy(w_h.at[base + q1], row1, sem1).wait()
        pltpu.make_async_copy(wc_h.at[base + q1], cnt1, sem1).wait()
        process(q1, row1, cnt1)
        return carry

    lax.fori_loop(0, QPW // 2, pair, jnp.int32(0))
    pltpu.sync_copy(stage, out_h.at[pl.ds(base, QPW)])


_bq = functools.partial(
    pl.kernel,
    compiler_params=_SC_PARAMS,
    out_type=jax.ShapeDtypeStruct((NPAD, NS), jnp.int32),
    mesh=plsc.VectorSubcoreMesh(core_axis_name="c", subcore_axis_name="s"),
    scratch_types=[
        pltpu.VMEM((NWORD,), jnp.float32),
        pltpu.VMEM((NWORD,), jnp.float32),
        pltpu.VMEM((NWC,), jnp.float32),
        pltpu.VMEM((NWC,), jnp.float32),
        pltpu.VMEM((16,), jnp.int32),
        pltpu.VMEM((16,), jnp.int32),
        pltpu.VMEM((NS,), jnp.int32),
        pltpu.VMEM((QPW, NS), jnp.int32),
        pltpu.SemaphoreType.DMA,
        pltpu.SemaphoreType.DMA,
    ],
)(_bq_body)


def _gather_body(tab_h, idx_h, out_h, idxv, rows0, rows1, sem0, sem1):
    wid = lax.axis_index("s") * 2 + lax.axis_index("c")
    pltpu.sync_copy(idx_h.at[wid], idxv)
    obase = wid * (GNC * GCH)
    pltpu.make_async_copy(tab_h.at[idxv.at[0]], rows0, sem0).start()

    def step(t, carry):
        j0 = 2 * t
        j1 = 2 * t + 1
        pltpu.make_async_copy(tab_h.at[idxv.at[j1]], rows1, sem1).start()
        pltpu.make_async_copy(tab_h.at[idxv.at[j0]], rows0, sem0).wait()
        pltpu.sync_copy(rows0, out_h.at[pl.ds(obase + j0 * GCH, GCH)])

        @pl.when(t < GNC // 2 - 1)
        def _():
            pltpu.make_async_copy(tab_h.at[idxv.at[j1 + 1]], rows0, sem0).start()

        pltpu.make_async_copy(tab_h.at[idxv.at[j1]], rows1, sem1).wait()
        pltpu.sync_copy(rows1, out_h.at[pl.ds(obase + j1 * GCH, GCH)])
        return carry

    lax.fori_loop(0, GNC // 2, step, jnp.int32(0))


_gather = functools.partial(
    pl.kernel,
    compiler_params=pltpu.CompilerParams(
        needs_layout_passes=False, use_tc_tiling_on_sc=False),
    out_type=jax.ShapeDtypeStruct((GROWS, CF), jnp.float32),
    mesh=plsc.VectorSubcoreMesh(core_axis_name="c", subcore_axis_name="s"),
    scratch_types=[
        pltpu.VMEM((GNC, GCH), jnp.int32),
        pltpu.VMEM((GCH, CF), jnp.float32),
        pltpu.VMEM((GCH, CF), jnp.float32),
        pltpu.SemaphoreType.DMA,
        pltpu.SemaphoreType.DMA,
    ],
)(_gather_body)


def _proj_body(t_ref, p_ref, w_ref, u_ref, c_ref):
    w = w_ref[...]
    u_ref[...] = jnp.dot(t_ref[...], w, preferred_element_type=jnp.float32)
    c_ref[...] = jnp.dot(p_ref[...], w[0:3, :], preferred_element_type=jnp.float32)


def _projk(T, Ppad, W1pad):
    return pl.pallas_call(
        _proj_body,
        grid=(5,),
        in_specs=[
            pl.BlockSpec((NPAD // 5, CIN), lambda r: (r, 0)),
            pl.BlockSpec((NPAD // 5, 3), lambda r: (r, 0)),
            pl.BlockSpec((CIN, CF), lambda r: (0, 0)),
        ],
        out_specs=[
            pl.BlockSpec((NPAD // 5, CF), lambda r: (r, 0)),
            pl.BlockSpec((NPAD // 5, CF), lambda r: (r, 0)),
        ],
        out_shape=[
            jax.ShapeDtypeStruct((NPAD, CF), jnp.float32),
            jax.ShapeDtypeStruct((NPAD, CF), jnp.float32),
        ],
    )(T, Ppad, W1pad)


def _mm1_body(g_ref, c_ref, prm_ref, y_ref, s_ref):
    g = pl.program_id(0)
    hw = g_ref[...]
    corr = c_ref[...]
    y = (hw.reshape(QBLK, NS, CF) - corr[:, None, :]).reshape(BLK, CF)
    y = y + prm_ref[0:1, :]
    y_ref[...] = y
    rid = lax.broadcasted_iota(jnp.int32, (BLK, 1), 0) + g * BLK
    ym = jnp.where(rid < VROWS, y, 0.0)

    @pl.when(g == 0)
    def _():
        s_ref[...] = jnp.zeros_like(s_ref)

    s_ref[0:1, :] += jnp.sum(ym, axis=0, keepdims=True)
    s_ref[1:2, :] += jnp.sum(ym * ym, axis=0, keepdims=True)


def _mm2_body(y1_ref, w_ref, prm_ref, y_ref, s_ref):
    g = pl.program_id(0)
    h1 = jnp.maximum(y1_ref[...] * prm_ref[0:1, :] + prm_ref[1:2, :], 0.0)
    y = jnp.dot(h1, w_ref[...], preferred_element_type=jnp.float32)
    y = y + prm_ref[2:3, :]
    y_ref[...] = y
    rid = lax.broadcasted_iota(jnp.int32, (BLK, 1), 0) + g * BLK
    ym = jnp.where(rid < VROWS, y, 0.0)

    @pl.when(g == 0)
    def _():
        s_ref[...] = jnp.zeros_like(s_ref)

    s_ref[0:1, :] += jnp.sum(ym, axis=0, keepdims=True)
    s_ref[1:2, :] += jnp.sum(ym * ym, axis=0, keepdims=True)


def _out_body(y2_ref, prm_ref, o_ref):
    h2 = jnp.maximum(y2_ref[...] * prm_ref[0:1, :] + prm_ref[1:2, :], 0.0)
    o_ref[...] = jnp.max(h2.reshape(QBLK, NS, CF), axis=1)


def _mlp1(G, C, prm1):
    return pl.pallas_call(
        _mm1_body,
        grid=(GRID,),
        in_specs=[
            pl.BlockSpec((BLK, CF), lambda g: (g, 0)),
            pl.BlockSpec((QBLK, CF), lambda g: (g, 0)),
            pl.BlockSpec((8, CF), lambda g: (0, 0)),
        ],
        out_specs=[
            pl.BlockSpec((BLK, CF), lambda g: (g, 0)),
            pl.BlockSpec((8, CF), lambda g: (0, 0)),
        ],
        out_shape=[
            jax.ShapeDtypeStruct((GROWS, CF), jnp.float32),
            jax.ShapeDtypeStruct((8, CF), jnp.float32),
        ],
    )(G, C, prm1)


def _mlp2(y1, W2, prm2):
    return pl.pallas_call(
        _mm2_body,
        grid=(GRID,),
        in_specs=[
            pl.BlockSpec((BLK, CF), lambda g: (g, 0)),
            pl.BlockSpec((CF, CF), lambda g: (0, 0)),
            pl.BlockSpec((8, CF), lambda g: (0, 0)),
        ],
        out_specs=[
            pl.BlockSpec((BLK, CF), lambda g: (g, 0)),
            pl.BlockSpec((8, CF), lambda g: (0, 0)),
        ],
        out_shape=[
            jax.ShapeDtypeStruct((GROWS, CF), jnp.float32),
            jax.ShapeDtypeStruct((8, CF), jnp.float32),
        ],
    )(y1, W2, prm2)


def _outk(y2, prm3):
    return pl.pallas_call(
        _out_body,
        grid=(GRID,),
        in_specs=[
            pl.BlockSpec((BLK, CF), lambda g: (g, 0)),
            pl.BlockSpec((8, CF), lambda g: (0, 0)),
        ],
        out_specs=pl.BlockSpec((QBLK, CF), lambda g: (g, 0)),
        out_shape=jax.ShapeDtypeStruct((NPAD, CF), jnp.float32),
    )(y2, prm3)


def kernel(p, x, W1, b1, g1, beta1, W2, b2, g2, beta2, b):
    f32 = jnp.float32
    padv = 1e6 + jnp.arange(N, NPAD, dtype=f32)
    Ppad = jnp.concatenate([p, jnp.stack([padv, padv, padv], axis=1)])
    jj = jnp.arange(CB)
    PK = jnp.where(jj[:, None] // 4 == jnp.arange(WBLK)[None, :],
                   (2.0 ** (jj % 4))[:, None], 0.0).astype(jnp.bfloat16)
    BW = jnp.where(jj[:, None] // 64 == jnp.arange(128)[None, :],
                   1.0, 0.0).astype(jnp.bfloat16)
    T = jnp.concatenate([p, x, jnp.zeros((N, CIN - 3 - CF), f32)], axis=1)
    T = jnp.concatenate(
        [T, T[N - 1:N], jnp.zeros((NPAD - N - 1, CIN), f32)], axis=0)
    W1pad = jnp.concatenate([W1, jnp.zeros((CIN - 3 - CF, CF), f32)])
    W, WC, U, C = _maskk(Ppad, Ppad.T, PK, BW, T, W1pad)
    idx_full = _bq(W, WC)
    idx_r = idx_full.reshape(NW, GNC, GCH)
    G = _gather(U, idx_r)

    prm1 = jnp.zeros((8, CF), f32).at[0].set(b1)
    y1, st1 = _mlp1(G, C, prm1)

    cnt = f32(VROWS)
    mu1 = st1[0] / cnt
    var1 = st1[1] / cnt - mu1 * mu1
    sc1 = g1 / jnp.sqrt(var1 + 1e-5)
    sh1 = beta1 - mu1 * sc1
    prm2 = jnp.zeros((8, CF), f32).at[0].set(sc1).at[1].set(sh1).at[2].set(b2)
    y2, st2 = _mlp2(y1, W2, prm2)

    mu2 = st2[0] / cnt
    var2 = st2[1] / cnt - mu2 * mu2
    sc2 = g2 / jnp.sqrt(var2 + 1e-5)
    sh2 = beta2 - mu2 * sc2
    prm3 = jnp.zeros((8, CF), f32).at[0].set(sc2).at[1].set(sh2)
    out = _outk(y2, prm3)
    return out[:N]


# speedup vs baseline: 1.1246x; 1.1246x over previous
import functools

import jax
import jax.numpy as jnp
from jax import lax
from jax.experimental import pallas as pl
from jax.experimental.pallas import tpu as pltpu
from jax.experimental.pallas import tpu_sc as plsc

N = 10000
NS = 16
R2 = 0.01
CF = 64
NW = 32
QPW = 320
NPAD = NW * QPW
NCH = NPAD // 16
CIN = 80
GROWS = NPAD * NS
VROWS = N * NS
BLK = 2048
QBLK = BLK // NS
GRID = GROWS // BLK
GCH = 128
GNC = GROWS // (NW * GCH)

RB = 512
CB = 1024

_SC_PARAMS = pltpu.CompilerParams(needs_layout_passes=False)


NWORD = NPAD // 4
NWIN = NPAD // 64
WBLK = NWORD // (NPAD // CB)
WCBLK = NWIN // (NPAD // CB)


def _sq_mask(pq_ref, pt_ref):
    pb = pq_ref[...]
    pt = pt_ref[...]
    pr2 = pb * pb
    pn_r = pr2[:, 0:1] + pr2[:, 1:2] + pr2[:, 2:3]
    pc2 = pt * pt
    pn_c = pc2[0:1, :] + pc2[1:2, :] + pc2[2:3, :]
    dot = jnp.dot(pb.astype(jnp.bfloat16), pt.astype(jnp.bfloat16),
                  preferred_element_type=jnp.float32)
    sq = (pn_r + pn_c) - 2.0 * dot
    return (sq <= R2).astype(jnp.bfloat16)


NWC = 1280


def _mask_body(pq_ref, pt_ref, pk_ref, bw_ref, w_ref, wc_ref):
    mb = _sq_mask(pq_ref, pt_ref)
    w_ref[...] = jnp.dot(mb, pk_ref[...], preferred_element_type=jnp.float32)
    wc_ref[...] = jnp.dot(mb, bw_ref[...], preferred_element_type=jnp.float32)


def _maskk(Ppad, PT, PK, BW):
    return pl.pallas_call(
        _mask_body,
        grid=(NPAD // RB, NPAD // CB),
        in_specs=[
            pl.BlockSpec((RB, 3), lambda r, c: (r, 0)),
            pl.BlockSpec((3, CB), lambda r, c: (0, c)),
            pl.BlockSpec((CB, WBLK), lambda r, c: (0, 0)),
            pl.BlockSpec((CB, 128), lambda r, c: (0, 0)),
        ],
        out_specs=[
            pl.BlockSpec((RB, WBLK), lambda r, c: (r, c)),
            pl.BlockSpec((RB, 128), lambda r, c: (r, c)),
        ],
        out_shape=[
            jax.ShapeDtypeStruct((NPAD, NWORD), jnp.float32),
            jax.ShapeDtypeStruct((NPAD, NWC), jnp.float32),
        ],
    )(Ppad, PT, PK, BW)


_REAL_GRPS = list(range(0, 80, 8))


def _bq_body(w_h, wc_h, out_h, row0, row1, cnt0, cnt1, wlist, cbase, buf,
             stage, sem0, sem1):
    wid = lax.axis_index("s") * 2 + lax.axis_index("c")
    lanes = lax.iota(jnp.int32, 16)
    base = wid * QPW
    wlist[...] = jnp.zeros((16,), jnp.int32)
    cbase[...] = jnp.zeros((16,), jnp.int32)
    pltpu.make_async_copy(w_h.at[base], row0, sem0).start()
    pltpu.make_async_copy(wc_h.at[base], cnt0, sem0).start()

    def process(q, rowv, cntv):
        run = jnp.zeros((16,), jnp.int32)
        nf = jnp.zeros((16,), jnp.int32)
        for g in _REAL_GRPS:
            wbase = (g // 8) * 16
            cwi = cntv[pl.ds(g * 16, 16)].astype(jnp.int32)
            cums = plsc.cumsum(cwi)
            cume = run + cums - cwi
            flag = (cwi > 0) & (cume < NS)
            fpos = nf + plsc.cumsum(flag.astype(jnp.int32)) - 1
            wm = flag & (fpos < 16)
            plsc.store_scatter(wlist, [fpos], wbase + lanes, mask=wm)
            plsc.store_scatter(cbase, [fpos], cume, mask=wm)
            nf = nf + plsc.all_reduce_population_count(flag)
            run = run + cums[jnp.zeros((16,), jnp.int32) + 15]
        wl = wlist[...]
        cb = cbase[...]

        for k in range(16):
            kv = nf > k
            w = wl[k]
            wi = rowv[pl.ds(w * 16, 16)].astype(jnp.int32)
            f0 = wi & 1
            f1 = (wi >> 1) & 1
            f2 = (wi >> 2) & 1
            f3 = (wi >> 3) & 1
            cwl = f0 + f1 + f2 + f3
            pexc = plsc.cumsum(cwl) - cwl
            cnd = w * 64 + 4 * lanes
            pos0 = cb[k] + pexc
            plsc.store_scatter(buf, [pos0], cnd,
                               mask=kv & (f0 > 0) & (pos0 < NS))
            pos1 = pos0 + f0
            plsc.store_scatter(buf, [pos1], cnd + 1,
                               mask=kv & (f1 > 0) & (pos1 < NS))
            pos2 = pos1 + f1
            plsc.store_scatter(buf, [pos2], cnd + 2,
                               mask=kv & (f2 > 0) & (pos2 < NS))
            pos3 = pos2 + f2
            plsc.store_scatter(buf, [pos3], cnd + 3,
                               mask=kv & (f3 > 0) & (pos3 < NS))

        vals = buf[...]
        first = jnp.where(run > 0, vals[jnp.zeros((16,), jnp.int32)], N)
        stage[q, :] = jnp.where(lanes < run, vals, first)

    def pair(t, carry):
        q0 = 2 * t
        q1 = 2 * t + 1
        pltpu.make_async_copy(w_h.at[base + q1], row1, sem1).start()
        pltpu.make_async_copy(wc_h.at[base + q1], cnt1, sem1).start()
        pltpu.make_async_copy(w_h.at[base + q0], row0, sem0).wait()
        pltpu.make_async_copy(wc_h.at[base + q0], cnt0, sem0).wait()
        process(q0, row0, cnt0)

        @pl.when(t < QPW // 2 - 1)
        def _():
            pltpu.make_async_copy(w_h.at[base + q1 + 1], row0, sem0).start()
            pltpu.make_async_copy(wc_h.at[base + q1 + 1], cnt0, sem0).start()

        pltpu.make_async_copy(w_h.at[base + q1], row1, sem1).wait()
        pltpu.make_async_copy(wc_h.at[base + q1], cnt1, sem1).wait()
        process(q1, row1, cnt1)
        return carry

    lax.fori_loop(0, QPW // 2, pair, jnp.int32(0))
    pltpu.sync_copy(stage, out_h.at[pl.ds(base, QPW)])


_bq = functools.partial(
    pl.kernel,
    compiler_params=_SC_PARAMS,
    out_type=jax.ShapeDtypeStruct((NPAD, NS), jnp.int32),
    mesh=plsc.VectorSubcoreMesh(core_axis_name="c", subcore_axis_name="s"),
    scratch_types=[
        pltpu.VMEM((NWORD,), jnp.float32),
        pltpu.VMEM((NWORD,), jnp.float32),
        pltpu.VMEM((NWC,), jnp.float32),
        pltpu.VMEM((NWC,), jnp.float32),
        pltpu.VMEM((16,), jnp.int32),
        pltpu.VMEM((16,), jnp.int32),
        pltpu.VMEM((NS,), jnp.int32),
        pltpu.VMEM((QPW, NS), jnp.int32),
        pltpu.SemaphoreType.DMA,
        pltpu.SemaphoreType.DMA,
    ],
)(_bq_body)


def _gather_body(tab_h, idx_h, out_h, idxv, rows0, rows1, sem0, sem1):
    wid = lax.axis_index("s") * 2 + lax.axis_index("c")
    pltpu.sync_copy(idx_h.at[wid], idxv)
    obase = wid * (GNC * GCH)
    pltpu.make_async_copy(tab_h.at[idxv.at[0]], rows0, sem0).start()

    def step(t, carry):
        j0 = 2 * t
        j1 = 2 * t + 1
        pltpu.make_async_copy(tab_h.at[idxv.at[j1]], rows1, sem1).start()
        pltpu.make_async_copy(tab_h.at[idxv.at[j0]], rows0, sem0).wait()
        pltpu.sync_copy(rows0, out_h.at[pl.ds(obase + j0 * GCH, GCH)])

        @pl.when(t < GNC // 2 - 1)
        def _():
            pltpu.make_async_copy(tab_h.at[idxv.at[j1 + 1]], rows0, sem0).start()

        pltpu.make_async_copy(tab_h.at[idxv.at[j1]], rows1, sem1).wait()
        pltpu.sync_copy(rows1, out_h.at[pl.ds(obase + j1 * GCH, GCH)])
        return carry

    lax.fori_loop(0, GNC // 2, step, jnp.int32(0))


_gather = functools.partial(
    pl.kernel,
    compiler_params=pltpu.CompilerParams(
        needs_layout_passes=False, use_tc_tiling_on_sc=False),
    out_type=jax.ShapeDtypeStruct((GROWS, CF), jnp.float32),
    mesh=plsc.VectorSubcoreMesh(core_axis_name="c", subcore_axis_name="s"),
    scratch_types=[
        pltpu.VMEM((GNC, GCH), jnp.int32),
        pltpu.VMEM((GCH, CF), jnp.float32),
        pltpu.VMEM((GCH, CF), jnp.float32),
        pltpu.SemaphoreType.DMA,
        pltpu.SemaphoreType.DMA,
    ],
)(_gather_body)


def _proj_body(t_ref, p_ref, w_ref, u_ref, c_ref):
    w = w_ref[...]
    u_ref[...] = jnp.dot(t_ref[...], w, preferred_element_type=jnp.float32)
    c_ref[...] = jnp.dot(p_ref[...], w[0:3, :], preferred_element_type=jnp.float32)


def _projk(T, Ppad, W1pad):
    return pl.pallas_call(
        _proj_body,
        grid=(5,),
        in_specs=[
            pl.BlockSpec((NPAD // 5, CIN), lambda r: (r, 0)),
            pl.BlockSpec((NPAD // 5, 3), lambda r: (r, 0)),
            pl.BlockSpec((CIN, CF), lambda r: (0, 0)),
        ],
        out_specs=[
            pl.BlockSpec((NPAD // 5, CF), lambda r: (r, 0)),
            pl.BlockSpec((NPAD // 5, CF), lambda r: (r, 0)),
        ],
        out_shape=[
            jax.ShapeDtypeStruct((NPAD, CF), jnp.float32),
            jax.ShapeDtypeStruct((NPAD, CF), jnp.float32),
        ],
    )(T, Ppad, W1pad)


def _mm1_body(g_ref, c_ref, prm_ref, y_ref, s_ref):
    g = pl.program_id(0)
    hw = g_ref[...]
    corr = c_ref[...]
    y = (hw.reshape(QBLK, NS, CF) - corr[:, None, :]).reshape(BLK, CF)
    y = y + prm_ref[0:1, :]
    y_ref[...] = y
    rid = lax.broadcasted_iota(jnp.int32, (BLK, 1), 0) + g * BLK
    ym = jnp.where(rid < VROWS, y, 0.0)

    @pl.when(g == 0)
    def _():
        s_ref[...] = jnp.zeros_like(s_ref)

    s_ref[0:1, :] += jnp.sum(ym, axis=0, keepdims=True)
    s_ref[1:2, :] += jnp.sum(ym * ym, axis=0, keepdims=True)


def _mm2_body(y1_ref, w_ref, prm_ref, y_ref, s_ref):
    g = pl.program_id(0)
    h1 = jnp.maximum(y1_ref[...] * prm_ref[0:1, :] + prm_ref[1:2, :], 0.0)
    y = jnp.dot(h1, w_ref[...], preferred_element_type=jnp.float32)
    y = y + prm_ref[2:3, :]
    y_ref[...] = y
    rid = lax.broadcasted_iota(jnp.int32, (BLK, 1), 0) + g * BLK
    ym = jnp.where(rid < VROWS, y, 0.0)

    @pl.when(g == 0)
    def _():
        s_ref[...] = jnp.zeros_like(s_ref)

    s_ref[0:1, :] += jnp.sum(ym, axis=0, keepdims=True)
    s_ref[1:2, :] += jnp.sum(ym * ym, axis=0, keepdims=True)


def _out_body(y2_ref, prm_ref, o_ref):
    h2 = jnp.maximum(y2_ref[...] * prm_ref[0:1, :] + prm_ref[1:2, :], 0.0)
    o_ref[...] = jnp.max(h2.reshape(QBLK, NS, CF), axis=1)


def _mlp1(G, C, prm1):
    return pl.pallas_call(
        _mm1_body,
        grid=(GRID,),
        in_specs=[
            pl.BlockSpec((BLK, CF), lambda g: (g, 0)),
            pl.BlockSpec((QBLK, CF), lambda g: (g, 0)),
            pl.BlockSpec((8, CF), lambda g: (0, 0)),
        ],
        out_specs=[
            pl.BlockSpec((BLK, CF), lambda g: (g, 0)),
            pl.BlockSpec((8, CF), lambda g: (0, 0)),
        ],
        out_shape=[
            jax.ShapeDtypeStruct((GROWS, CF), jnp.float32),
            jax.ShapeDtypeStruct((8, CF), jnp.float32),
        ],
    )(G, C, prm1)


def _mlp2(y1, W2, prm2):
    return pl.pallas_call(
        _mm2_body,
        grid=(GRID,),
        in_specs=[
            pl.BlockSpec((BLK, CF), lambda g: (g, 0)),
            pl.BlockSpec((CF, CF), lambda g: (0, 0)),
            pl.BlockSpec((8, CF), lambda g: (0, 0)),
        ],
        out_specs=[
            pl.BlockSpec((BLK, CF), lambda g: (g, 0)),
            pl.BlockSpec((8, CF), lambda g: (0, 0)),
        ],
        out_shape=[
            jax.ShapeDtypeStruct((GROWS, CF), jnp.float32),
            jax.ShapeDtypeStruct((8, CF), jnp.float32),
        ],
    )(y1, W2, prm2)


def _outk(y2, prm3):
    return pl.pallas_call(
        _out_body,
        grid=(GRID,),
        in_specs=[
            pl.BlockSpec((BLK, CF), lambda g: (g, 0)),
            pl.BlockSpec((8, CF), lambda g: (0, 0)),
        ],
        out_specs=pl.BlockSpec((QBLK, CF), lambda g: (g, 0)),
        out_shape=jax.ShapeDtypeStruct((NPAD, CF), jnp.float32),
    )(y2, prm3)


def kernel(p, x, W1, b1, g1, beta1, W2, b2, g2, beta2, b):
    f32 = jnp.float32
    padv = 1e6 + jnp.arange(N, NPAD, dtype=f32)
    Ppad = jnp.concatenate([p, jnp.stack([padv, padv, padv], axis=1)])
    jj = jnp.arange(CB)
    PK = jnp.where(jj[:, None] // 4 == jnp.arange(WBLK)[None, :],
                   (2.0 ** (jj % 4))[:, None], 0.0).astype(jnp.bfloat16)
    BW = jnp.where(jj[:, None] // 64 == jnp.arange(128)[None, :],
                   1.0, 0.0).astype(jnp.bfloat16)
    T = jnp.concatenate([p, x, jnp.zeros((N, CIN - 3 - CF), f32)], axis=1)
    T = jnp.concatenate(
        [T, T[N - 1:N], jnp.zeros((NPAD - N - 1, CIN), f32)], axis=0)
    W1pad = jnp.concatenate([W1, jnp.zeros((CIN - 3 - CF, CF), f32)])
    W, WC = _maskk(Ppad, Ppad.T, PK, BW)
    idx_full = _bq(W, WC)
    idx_r = idx_full.reshape(NW, GNC, GCH)
    Pq = jnp.concatenate([p, jnp.zeros((NPAD - N, 3), f32)])
    U, C = _projk(T, Pq, W1pad)
    G = _gather(U, idx_r)

    prm1 = jnp.zeros((8, CF), f32).at[0].set(b1)
    y1, st1 = _mlp1(G, C, prm1)

    cnt = f32(VROWS)
    mu1 = st1[0] / cnt
    var1 = st1[1] / cnt - mu1 * mu1
    sc1 = g1 / jnp.sqrt(var1 + 1e-5)
    sh1 = beta1 - mu1 * sc1
    prm2 = jnp.zeros((8, CF), f32).at[0].set(sc1).at[1].set(sh1).at[2].set(b2)
    y2, st2 = _mlp2(y1, W2, prm2)

    mu2 = st2[0] / cnt
    var2 = st2[1] / cnt - mu2 * mu2
    sc2 = g2 / jnp.sqrt(var2 + 1e-5)
    sh2 = beta2 - mu2 * sc2
    prm3 = jnp.zeros((8, CF), f32).at[0].set(sc2).at[1].set(sh2)
    out = _outk(y2, prm3)
    return out[:N]


# speedup vs baseline: 1.1709x; 1.0412x over previous
import functools

import jax
import jax.numpy as jnp
from jax import lax
from jax.experimental import pallas as pl
from jax.experimental.pallas import tpu as pltpu
from jax.experimental.pallas import tpu_sc as plsc

N = 10000
NS = 16
R2 = 0.01
CF = 64
NW = 32
QPW = 320
NPAD = NW * QPW
NCH = NPAD // 16
CIN = 80
GROWS = NPAD * NS
VROWS = N * NS
BLK = 2048
QBLK = BLK // NS
GRID = GROWS // BLK
GCH = 128
GNC = GROWS // (NW * GCH)

RB = 512
CB = 1024

_SC_PARAMS = pltpu.CompilerParams(needs_layout_passes=False)


NWORD = NPAD // 4
NWIN = NPAD // 64
WBLK = NWORD // (NPAD // CB)
WCBLK = NWIN // (NPAD // CB)


def _sq_mask(pq_ref, pt_ref):
    pb = pq_ref[...]
    pt = pt_ref[...]
    pr2 = pb * pb
    pn_r = pr2[:, 0:1] + pr2[:, 1:2] + pr2[:, 2:3]
    pc2 = pt * pt
    pn_c = pc2[0:1, :] + pc2[1:2, :] + pc2[2:3, :]
    dot = jnp.dot(pb.astype(jnp.bfloat16), pt.astype(jnp.bfloat16),
                  preferred_element_type=jnp.float32)
    sq = (pn_r + pn_c) - 2.0 * dot
    return (sq <= R2).astype(jnp.bfloat16)


NWC = 1280


def _mask_body(pq_ref, pt_ref, pk_ref, bw_ref, w_ref, wc_ref):
    mb = _sq_mask(pq_ref, pt_ref)
    w_ref[...] = jnp.dot(mb, pk_ref[...], preferred_element_type=jnp.float32)
    wc_ref[...] = jnp.dot(mb, bw_ref[...], preferred_element_type=jnp.float32)


def _maskk(Ppad, PT, PK, BW):
    return pl.pallas_call(
        _mask_body,
        grid=(NPAD // RB, NPAD // CB),
        in_specs=[
            pl.BlockSpec((RB, 3), lambda r, c: (r, 0)),
            pl.BlockSpec((3, CB), lambda r, c: (0, c)),
            pl.BlockSpec((CB, WBLK), lambda r, c: (0, 0)),
            pl.BlockSpec((CB, 128), lambda r, c: (0, 0)),
        ],
        out_specs=[
            pl.BlockSpec((RB, WBLK), lambda r, c: (r, c)),
            pl.BlockSpec((RB, 128), lambda r, c: (r, c)),
        ],
        out_shape=[
            jax.ShapeDtypeStruct((NPAD, NWORD), jnp.float32),
            jax.ShapeDtypeStruct((NPAD, NWC), jnp.float32),
        ],
    )(Ppad, PT, PK, BW)


_REAL_GRPS = list(range(0, 80, 8))


def _bq_body(w_h, wc_h, out_h, row0, row1, cnt0, cnt1, wlist, cbase, buf,
             stage, sem0, sem1):
    wid = lax.axis_index("s") * 2 + lax.axis_index("c")
    lanes = lax.iota(jnp.int32, 16)
    base = wid * QPW
    wlist[...] = jnp.zeros((16,), jnp.int32)
    cbase[...] = jnp.zeros((16,), jnp.int32)
    pltpu.make_async_copy(w_h.at[base], row0, sem0).start()
    pltpu.make_async_copy(wc_h.at[base], cnt0, sem0).start()

    def process(q, rowv, cntv):
        run = jnp.zeros((16,), jnp.int32)
        nf = jnp.zeros((16,), jnp.int32)
        for g in _REAL_GRPS:
            wbase = (g // 8) * 16
            cwi = cntv[pl.ds(g * 16, 16)].astype(jnp.int32)
            cums = plsc.cumsum(cwi)
            cume = run + cums - cwi
            flag = (cwi > 0) & (cume < NS)
            fpos = nf + plsc.cumsum(flag.astype(jnp.int32)) - 1
            wm = flag & (fpos < 16)
            plsc.store_scatter(wlist, [fpos], wbase + lanes, mask=wm)
            plsc.store_scatter(cbase, [fpos], cume, mask=wm)
            nf = nf + plsc.all_reduce_population_count(flag)
            run = run + cums[jnp.zeros((16,), jnp.int32) + 15]
        wl = wlist[...]
        cb = cbase[...]

        for k in range(16):
            kv = nf > k
            w = wl[k]
            wi = rowv[pl.ds(w * 16, 16)].astype(jnp.int32)
            f0 = wi & 1
            f1 = (wi >> 1) & 1
            f2 = (wi >> 2) & 1
            f3 = (wi >> 3) & 1
            cwl = f0 + f1 + f2 + f3
            pexc = plsc.cumsum(cwl) - cwl
            cnd = w * 64 + 4 * lanes
            pos0 = cb[k] + pexc
            plsc.store_scatter(buf, [pos0], cnd,
                               mask=kv & (f0 > 0) & (pos0 < NS))
            pos1 = pos0 + f0
            plsc.store_scatter(buf, [pos1], cnd + 1,
                               mask=kv & (f1 > 0) & (pos1 < NS))
            pos2 = pos1 + f1
            plsc.store_scatter(buf, [pos2], cnd + 2,
                               mask=kv & (f2 > 0) & (pos2 < NS))
            pos3 = pos2 + f2
            plsc.store_scatter(buf, [pos3], cnd + 3,
                               mask=kv & (f3 > 0) & (pos3 < NS))

        vals = buf[...]
        first = jnp.where(run > 0, vals[jnp.zeros((16,), jnp.int32)], N)
        stage[q, :] = jnp.where(lanes < run, vals, first)

    def pair(t, carry):
        q0 = 2 * t
        q1 = 2 * t + 1
        pltpu.make_async_copy(w_h.at[base + q1], row1, sem1).start()
        pltpu.make_async_copy(wc_h.at[base + q1], cnt1, sem1).start()
        pltpu.make_async_copy(w_h.at[base + q0], row0, sem0).wait()
        pltpu.make_async_copy(wc_h.at[base + q0], cnt0, sem0).wait()
        process(q0, row0, cnt0)

        @pl.when(t < QPW // 2 - 1)
        def _():
            pltpu.make_async_copy(w_h.at[base + q1 + 1], row0, sem0).start()
            pltpu.make_async_copy(wc_h.at[base + q1 + 1], cnt0, sem0).start()

        pltpu.make_async_copy(w_h.at[base + q1], row1, sem1).wait()
        pltpu.make_async_copy(wc_h.at[base + q1], cnt1, sem1).wait()
        process(q1, row1, cnt1)
        return carry

    lax.fori_loop(0, QPW // 2, pair, jnp.int32(0))
    pltpu.sync_copy(stage, out_h.at[pl.ds(base, QPW)])


_bq = functools.partial(
    pl.kernel,
    compiler_params=_SC_PARAMS,
    out_type=jax.ShapeDtypeStruct((NPAD, NS), jnp.int32),
    mesh=plsc.VectorSubcoreMesh(core_axis_name="c", subcore_axis_name="s"),
    scratch_types=[
        pltpu.VMEM((NWORD,), jnp.float32),
        pltpu.VMEM((NWORD,), jnp.float32),
        pltpu.VMEM((NWC,), jnp.float32),
        pltpu.VMEM((NWC,), jnp.float32),
        pltpu.VMEM((16,), jnp.int32),
        pltpu.VMEM((16,), jnp.int32),
        pltpu.VMEM((NS,), jnp.int32),
        pltpu.VMEM((QPW, NS), jnp.int32),
        pltpu.SemaphoreType.DMA,
        pltpu.SemaphoreType.DMA,
    ],
)(_bq_body)


def _gather_body(tab_h, idx_h, out_h, idxv, rows0, rows1, sem0, sem1):
    wid = lax.axis_index("s") * 2 + lax.axis_index("c")
    pltpu.sync_copy(idx_h.at[wid], idxv)
    obase = wid * (GNC * GCH)
    pltpu.make_async_copy(tab_h.at[idxv.at[0]], rows0, sem0).start()

    def step(t, carry):
        j0 = 2 * t
        j1 = 2 * t + 1
        pltpu.make_async_copy(tab_h.at[idxv.at[j1]], rows1, sem1).start()
        pltpu.make_async_copy(tab_h.at[idxv.at[j0]], rows0, sem0).wait()
        pltpu.sync_copy(rows0, out_h.at[pl.ds(obase + j0 * GCH, GCH)])

        @pl.when(t < GNC // 2 - 1)
        def _():
            pltpu.make_async_copy(tab_h.at[idxv.at[j1 + 1]], rows0, sem0).start()

        pltpu.make_async_copy(tab_h.at[idxv.at[j1]], rows1, sem1).wait()
        pltpu.sync_copy(rows1, out_h.at[pl.ds(obase + j1 * GCH, GCH)])
        return carry

    lax.fori_loop(0, GNC // 2, step, jnp.int32(0))


_gather = functools.partial(
    pl.kernel,
    compiler_params=pltpu.CompilerParams(
        needs_layout_passes=False, use_tc_tiling_on_sc=False),
    out_type=jax.ShapeDtypeStruct((GROWS, CF), jnp.float32),
    mesh=plsc.VectorSubcoreMesh(core_axis_name="c", subcore_axis_name="s"),
    scratch_types=[
        pltpu.VMEM((GNC, GCH), jnp.int32),
        pltpu.VMEM((GCH, CF), jnp.float32),
        pltpu.VMEM((GCH, CF), jnp.float32),
        pltpu.SemaphoreType.DMA,
        pltpu.SemaphoreType.DMA,
    ],
)(_gather_body)


def _proj_body(t_ref, p_ref, w_ref, u_ref, c_ref):
    w = w_ref[...]
    u_ref[...] = jnp.dot(t_ref[...], w, preferred_element_type=jnp.float32)
    c_ref[...] = jnp.dot(p_ref[...], w[0:3, :], preferred_element_type=jnp.float32)


def _projk(T, Ppad, W1pad):
    return pl.pallas_call(
        _proj_body,
        grid=(5,),
        in_specs=[
            pl.BlockSpec((NPAD // 5, CIN), lambda r: (r, 0)),
            pl.BlockSpec((NPAD // 5, 3), lambda r: (r, 0)),
            pl.BlockSpec((CIN, CF), lambda r: (0, 0)),
        ],
        out_specs=[
            pl.BlockSpec((NPAD // 5, CF), lambda r: (r, 0)),
            pl.BlockSpec((NPAD // 5, CF), lambda r: (r, 0)),
        ],
        out_shape=[
            jax.ShapeDtypeStruct((NPAD, CF), jnp.float32),
            jax.ShapeDtypeStruct((NPAD, CF), jnp.float32),
        ],
    )(T, Ppad, W1pad)


def _mm1_body(g_ref, c_ref, prm_ref, y_ref, s_ref):
    g = pl.program_id(0)
    hw = g_ref[...]
    corr = c_ref[...]
    y = (hw.reshape(QBLK, NS, CF) - corr[:, None, :]).reshape(BLK, CF)
    y = y + prm_ref[0:1, :]
    y_ref[...] = y.astype(jnp.bfloat16)
    rid = lax.broadcasted_iota(jnp.int32, (BLK, 1), 0) + g * BLK
    ym = jnp.where(rid < VROWS, y, 0.0)

    @pl.when(g == 0)
    def _():
        s_ref[...] = jnp.zeros_like(s_ref)

    s_ref[0:1, :] += jnp.sum(ym, axis=0, keepdims=True)
    s_ref[1:2, :] += jnp.sum(ym * ym, axis=0, keepdims=True)


def _mm2_body(y1_ref, w_ref, prm_ref, y_ref, s_ref):
    g = pl.program_id(0)
    h1 = jnp.maximum(
        y1_ref[...].astype(jnp.float32) * prm_ref[0:1, :] + prm_ref[1:2, :],
        0.0)
    y = jnp.dot(h1, w_ref[...], preferred_element_type=jnp.float32)
    y = y + prm_ref[2:3, :]
    y_ref[...] = y.astype(jnp.bfloat16)
    rid = lax.broadcasted_iota(jnp.int32, (BLK, 1), 0) + g * BLK
    ym = jnp.where(rid < VROWS, y, 0.0)

    @pl.when(g == 0)
    def _():
        s_ref[...] = jnp.zeros_like(s_ref)

    s_ref[0:1, :] += jnp.sum(ym, axis=0, keepdims=True)
    s_ref[1:2, :] += jnp.sum(ym * ym, axis=0, keepdims=True)


def _out_body(y2_ref, prm_ref, o_ref):
    h2 = jnp.maximum(
        y2_ref[...].astype(jnp.float32) * prm_ref[0:1, :] + prm_ref[1:2, :],
        0.0)
    o_ref[...] = jnp.max(h2.reshape(QBLK, NS, CF), axis=1)


def _mlp1(G, C, prm1):
    return pl.pallas_call(
        _mm1_body,
        grid=(GRID,),
        in_specs=[
            pl.BlockSpec((BLK, CF), lambda g: (g, 0)),
            pl.BlockSpec((QBLK, CF), lambda g: (g, 0)),
            pl.BlockSpec((8, CF), lambda g: (0, 0)),
        ],
        out_specs=[
            pl.BlockSpec((BLK, CF), lambda g: (g, 0)),
            pl.BlockSpec((8, CF), lambda g: (0, 0)),
        ],
        out_shape=[
            jax.ShapeDtypeStruct((GROWS, CF), jnp.bfloat16),
            jax.ShapeDtypeStruct((8, CF), jnp.float32),
        ],
    )(G, C, prm1)


def _mlp2(y1, W2, prm2):
    return pl.pallas_call(
        _mm2_body,
        grid=(GRID,),
        in_specs=[
            pl.BlockSpec((BLK, CF), lambda g: (g, 0)),
            pl.BlockSpec((CF, CF), lambda g: (0, 0)),
            pl.BlockSpec((8, CF), lambda g: (0, 0)),
        ],
        out_specs=[
            pl.BlockSpec((BLK, CF), lambda g: (g, 0)),
            pl.BlockSpec((8, CF), lambda g: (0, 0)),
        ],
        out_shape=[
            jax.ShapeDtypeStruct((GROWS, CF), jnp.bfloat16),
            jax.ShapeDtypeStruct((8, CF), jnp.float32),
        ],
    )(y1, W2, prm2)


def _outk(y2, prm3):
    return pl.pallas_call(
        _out_body,
        grid=(GRID,),
        in_specs=[
            pl.BlockSpec((BLK, CF), lambda g: (g, 0)),
            pl.BlockSpec((8, CF), lambda g: (0, 0)),
        ],
        out_specs=pl.BlockSpec((QBLK, CF), lambda g: (g, 0)),
        out_shape=jax.ShapeDtypeStruct((NPAD, CF), jnp.float32),
    )(y2, prm3)


def kernel(p, x, W1, b1, g1, beta1, W2, b2, g2, beta2, b):
    f32 = jnp.float32
    padv = 1e6 + jnp.arange(N, NPAD, dtype=f32)
    Ppad = jnp.concatenate([p, jnp.stack([padv, padv, padv], axis=1)])
    jj = jnp.arange(CB)
    PK = jnp.where(jj[:, None] // 4 == jnp.arange(WBLK)[None, :],
                   (2.0 ** (jj % 4))[:, None], 0.0).astype(jnp.bfloat16)
    BW = jnp.where(jj[:, None] // 64 == jnp.arange(128)[None, :],
                   1.0, 0.0).astype(jnp.bfloat16)
    T = jnp.concatenate([p, x, jnp.zeros((N, CIN - 3 - CF), f32)], axis=1)
    T = jnp.concatenate(
        [T, T[N - 1:N], jnp.zeros((NPAD - N - 1, CIN), f32)], axis=0)
    W1pad = jnp.concatenate([W1, jnp.zeros((CIN - 3 - CF, CF), f32)])
    W, WC = _maskk(Ppad, Ppad.T, PK, BW)
    idx_full = _bq(W, WC)
    idx_r = idx_full.reshape(NW, GNC, GCH)
    Pq = jnp.concatenate([p, jnp.zeros((NPAD - N, 3), f32)])
    U, C = _projk(T, Pq, W1pad)
    G = _gather(U, idx_r)

    prm1 = jnp.zeros((8, CF), f32).at[0].set(b1)
    y1, st1 = _mlp1(G, C, prm1)

    cnt = f32(VROWS)
    mu1 = st1[0] / cnt
    var1 = st1[1] / cnt - mu1 * mu1
    sc1 = g1 / jnp.sqrt(var1 + 1e-5)
    sh1 = beta1 - mu1 * sc1
    prm2 = jnp.zeros((8, CF), f32).at[0].set(sc1).at[1].set(sh1).at[2].set(b2)
    y2, st2 = _mlp2(y1, W2, prm2)

    mu2 = st2[0] / cnt
    var2 = st2[1] / cnt - mu2 * mu2
    sc2 = g2 / jnp.sqrt(var2 + 1e-5)
    sh2 = beta2 - mu2 * sc2
    prm3 = jnp.zeros((8, CF), f32).at[0].set(sc2).at[1].set(sh2)
    out = _outk(y2, prm3)
    return out[:N]


# speedup vs baseline: 1.3104x; 1.1191x over previous
import functools

import jax
import jax.numpy as jnp
from jax import lax
from jax.experimental import pallas as pl
from jax.experimental.pallas import tpu as pltpu
from jax.experimental.pallas import tpu_sc as plsc

N = 10000
NS = 16
R2 = 0.01
CF = 64
NW = 32
QPW = 320
NPAD = NW * QPW
NCH = NPAD // 16
CIN = 80
GROWS = NPAD * NS
VROWS = N * NS
BLK = 2048
QBLK = BLK // NS
GRID = GROWS // BLK
GCH = 128
GNC = GROWS // (NW * GCH)

RB = 512
CB = 1024

_SC_PARAMS = pltpu.CompilerParams(needs_layout_passes=False)


NWORD = NPAD // 4
NWIN = NPAD // 64
WBLK = NWORD // (NPAD // CB)
WCBLK = NWIN // (NPAD // CB)


def _sq_mask(pq_ref, pt_ref):
    pb = pq_ref[...]
    pt = pt_ref[...]
    pr2 = pb * pb
    pn_r = pr2[:, 0:1] + pr2[:, 1:2] + pr2[:, 2:3]
    pc2 = pt * pt
    pn_c = pc2[0:1, :] + pc2[1:2, :] + pc2[2:3, :]
    dot = jnp.dot(pb.astype(jnp.bfloat16), pt.astype(jnp.bfloat16),
                  preferred_element_type=jnp.float32)
    sq = (pn_r + pn_c) - 2.0 * dot
    return (sq <= R2).astype(jnp.bfloat16)


NWC = 1280


def _mask_body(pq_ref, pt_ref, pk_ref, bw_ref, w_ref, wc_ref):
    mb = _sq_mask(pq_ref, pt_ref)
    w_ref[...] = jnp.dot(mb, pk_ref[...], preferred_element_type=jnp.float32)
    wc_ref[...] = jnp.dot(mb, bw_ref[...], preferred_element_type=jnp.float32)


def _maskk(Ppad, PT, PK, BW):
    rows = Ppad.shape[0]
    return pl.pallas_call(
        _mask_body,
        grid=(rows // RB, NPAD // CB),
        in_specs=[
            pl.BlockSpec((RB, 3), lambda r, c: (r, 0)),
            pl.BlockSpec((3, CB), lambda r, c: (0, c)),
            pl.BlockSpec((CB, WBLK), lambda r, c: (0, 0)),
            pl.BlockSpec((CB, 128), lambda r, c: (0, 0)),
        ],
        out_specs=[
            pl.BlockSpec((RB, WBLK), lambda r, c: (r, c)),
            pl.BlockSpec((RB, 128), lambda r, c: (r, c)),
        ],
        out_shape=[
            jax.ShapeDtypeStruct((rows, NWORD), jnp.float32),
            jax.ShapeDtypeStruct((rows, NWC), jnp.float32),
        ],
    )(Ppad, PT, PK, BW)


_REAL_GRPS = list(range(0, 80, 8))


def _make_bq_body(qpw):
  def _bq_body(w_h, wc_h, out_h, row0, row1, cnt0, cnt1, wlist, cbase, buf,
               stage, sem0, sem1):
    wid = lax.axis_index("s") * 2 + lax.axis_index("c")
    lanes = lax.iota(jnp.int32, 16)
    base = wid * qpw
    wlist[...] = jnp.zeros((16,), jnp.int32)
    cbase[...] = jnp.zeros((16,), jnp.int32)
    pltpu.make_async_copy(w_h.at[base], row0, sem0).start()
    pltpu.make_async_copy(wc_h.at[base], cnt0, sem0).start()

    def process(q, rowv, cntv):
        run = jnp.zeros((16,), jnp.int32)
        nf = jnp.zeros((16,), jnp.int32)
        for g in _REAL_GRPS:
            wbase = (g // 8) * 16
            cwi = cntv[pl.ds(g * 16, 16)].astype(jnp.int32)
            cums = plsc.cumsum(cwi)
            cume = run + cums - cwi
            flag = (cwi > 0) & (cume < NS)
            fpos = nf + plsc.cumsum(flag.astype(jnp.int32)) - 1
            wm = flag & (fpos < 16)
            plsc.store_scatter(wlist, [fpos], wbase + lanes, mask=wm)
            plsc.store_scatter(cbase, [fpos], cume, mask=wm)
            nf = nf + plsc.all_reduce_population_count(flag)
            run = run + cums[jnp.zeros((16,), jnp.int32) + 15]
        wl = wlist[...]
        cb = cbase[...]

        for k in range(16):
            kv = nf > k
            w = wl[k]
            wi = rowv[pl.ds(w * 16, 16)].astype(jnp.int32)
            f0 = wi & 1
            f1 = (wi >> 1) & 1
            f2 = (wi >> 2) & 1
            f3 = (wi >> 3) & 1
            cwl = f0 + f1 + f2 + f3
            pexc = plsc.cumsum(cwl) - cwl
            cnd = w * 64 + 4 * lanes
            pos0 = cb[k] + pexc
            plsc.store_scatter(buf, [pos0], cnd,
                               mask=kv & (f0 > 0) & (pos0 < NS))
            pos1 = pos0 + f0
            plsc.store_scatter(buf, [pos1], cnd + 1,
                               mask=kv & (f1 > 0) & (pos1 < NS))
            pos2 = pos1 + f1
            plsc.store_scatter(buf, [pos2], cnd + 2,
                               mask=kv & (f2 > 0) & (pos2 < NS))
            pos3 = pos2 + f2
            plsc.store_scatter(buf, [pos3], cnd + 3,
                               mask=kv & (f3 > 0) & (pos3 < NS))

        vals = buf[...]
        first = jnp.where(run > 0, vals[jnp.zeros((16,), jnp.int32)], N)
        stage[q, :] = jnp.where(lanes < run, vals, first)

    def pair(t, carry):
        q0 = 2 * t
        q1 = 2 * t + 1
        pltpu.make_async_copy(w_h.at[base + q1], row1, sem1).start()
        pltpu.make_async_copy(wc_h.at[base + q1], cnt1, sem1).start()
        pltpu.make_async_copy(w_h.at[base + q0], row0, sem0).wait()
        pltpu.make_async_copy(wc_h.at[base + q0], cnt0, sem0).wait()
        process(q0, row0, cnt0)

        @pl.when(t < qpw // 2 - 1)
        def _():
            pltpu.make_async_copy(w_h.at[base + q1 + 1], row0, sem0).start()
            pltpu.make_async_copy(wc_h.at[base + q1 + 1], cnt0, sem0).start()

        pltpu.make_async_copy(w_h.at[base + q1], row1, sem1).wait()
        pltpu.make_async_copy(wc_h.at[base + q1], cnt1, sem1).wait()
        process(q1, row1, cnt1)
        return carry

    lax.fori_loop(0, qpw // 2, pair, jnp.int32(0))
    pltpu.sync_copy(stage, out_h.at[pl.ds(base, qpw)])

  return _bq_body


def _make_bq(qpw):
    rows = qpw * NW
    return functools.partial(
        pl.kernel,
        compiler_params=_SC_PARAMS,
        out_type=jax.ShapeDtypeStruct((rows, NS), jnp.int32),
        mesh=plsc.VectorSubcoreMesh(core_axis_name="c", subcore_axis_name="s"),
        scratch_types=[
            pltpu.VMEM((NWORD,), jnp.float32),
            pltpu.VMEM((NWORD,), jnp.float32),
            pltpu.VMEM((NWC,), jnp.float32),
            pltpu.VMEM((NWC,), jnp.float32),
            pltpu.VMEM((16,), jnp.int32),
            pltpu.VMEM((16,), jnp.int32),
            pltpu.VMEM((NS,), jnp.int32),
            pltpu.VMEM((qpw, NS), jnp.int32),
            pltpu.SemaphoreType.DMA,
            pltpu.SemaphoreType.DMA,
        ],
    )(_make_bq_body(qpw))


_bq_half = _make_bq(QPW // 2)


def _gather_body(tab_h, idx_h, out_h, idxv, rows0, rows1, sem0, sem1):
    wid = lax.axis_index("s") * 2 + lax.axis_index("c")
    pltpu.sync_copy(idx_h.at[wid], idxv)
    obase = wid * (GNC * GCH)
    pltpu.make_async_copy(tab_h.at[idxv.at[0]], rows0, sem0).start()

    def step(t, carry):
        j0 = 2 * t
        j1 = 2 * t + 1
        pltpu.make_async_copy(tab_h.at[idxv.at[j1]], rows1, sem1).start()
        pltpu.make_async_copy(tab_h.at[idxv.at[j0]], rows0, sem0).wait()
        pltpu.sync_copy(rows0, out_h.at[pl.ds(obase + j0 * GCH, GCH)])

        @pl.when(t < GNC // 2 - 1)
        def _():
            pltpu.make_async_copy(tab_h.at[idxv.at[j1 + 1]], rows0, sem0).start()

        pltpu.make_async_copy(tab_h.at[idxv.at[j1]], rows1, sem1).wait()
        pltpu.sync_copy(rows1, out_h.at[pl.ds(obase + j1 * GCH, GCH)])
        return carry

    lax.fori_loop(0, GNC // 2, step, jnp.int32(0))


_gather = functools.partial(
    pl.kernel,
    compiler_params=pltpu.CompilerParams(
        needs_layout_passes=False, use_tc_tiling_on_sc=False),
    out_type=jax.ShapeDtypeStruct((GROWS, CF), jnp.float32),
    mesh=plsc.VectorSubcoreMesh(core_axis_name="c", subcore_axis_name="s"),
    scratch_types=[
        pltpu.VMEM((GNC, GCH), jnp.int32),
        pltpu.VMEM((GCH, CF), jnp.float32),
        pltpu.VMEM((GCH, CF), jnp.float32),
        pltpu.SemaphoreType.DMA,
        pltpu.SemaphoreType.DMA,
    ],
)(_gather_body)


def _proj_body(t_ref, p_ref, w_ref, u_ref, c_ref):
    w = w_ref[...]
    u_ref[...] = jnp.dot(t_ref[...], w, preferred_element_type=jnp.float32)
    c_ref[...] = jnp.dot(p_ref[...], w[0:3, :], preferred_element_type=jnp.float32)


def _projk(T, Ppad, W1pad):
    return pl.pallas_call(
        _proj_body,
        grid=(5,),
        in_specs=[
            pl.BlockSpec((NPAD // 5, CIN), lambda r: (r, 0)),
            pl.BlockSpec((NPAD // 5, 3), lambda r: (r, 0)),
            pl.BlockSpec((CIN, CF), lambda r: (0, 0)),
        ],
        out_specs=[
            pl.BlockSpec((NPAD // 5, CF), lambda r: (r, 0)),
            pl.BlockSpec((NPAD // 5, CF), lambda r: (r, 0)),
        ],
        out_shape=[
            jax.ShapeDtypeStruct((NPAD, CF), jnp.float32),
            jax.ShapeDtypeStruct((NPAD, CF), jnp.float32),
        ],
    )(T, Ppad, W1pad)


def _mm1_body(g_ref, c_ref, prm_ref, y_ref, s_ref):
    g = pl.program_id(0)
    hw = g_ref[...]
    corr = c_ref[...]
    y = (hw.reshape(QBLK, NS, CF) - corr[:, None, :]).reshape(BLK, CF)
    y = y + prm_ref[0:1, :]
    y_ref[...] = y.astype(jnp.bfloat16)
    rid = lax.broadcasted_iota(jnp.int32, (BLK, 1), 0) + g * BLK
    ym = jnp.where(rid < VROWS, y, 0.0)

    @pl.when(g == 0)
    def _():
        s_ref[...] = jnp.zeros_like(s_ref)

    s_ref[0:1, :] += jnp.sum(ym, axis=0, keepdims=True)
    s_ref[1:2, :] += jnp.sum(ym * ym, axis=0, keepdims=True)


def _mm2_body(y1_ref, w_ref, prm_ref, y_ref, s_ref):
    g = pl.program_id(0)
    h1 = jnp.maximum(
        y1_ref[...].astype(jnp.float32) * prm_ref[0:1, :] + prm_ref[1:2, :],
        0.0)
    y = jnp.dot(h1, w_ref[...], preferred_element_type=jnp.float32)
    y = y + prm_ref[2:3, :]
    y_ref[...] = y.astype(jnp.bfloat16)
    rid = lax.broadcasted_iota(jnp.int32, (BLK, 1), 0) + g * BLK
    ym = jnp.where(rid < VROWS, y, 0.0)

    @pl.when(g == 0)
    def _():
        s_ref[...] = jnp.zeros_like(s_ref)

    s_ref[0:1, :] += jnp.sum(ym, axis=0, keepdims=True)
    s_ref[1:2, :] += jnp.sum(ym * ym, axis=0, keepdims=True)


def _out_body(y2_ref, prm_ref, o_ref):
    h2 = jnp.maximum(
        y2_ref[...].astype(jnp.float32) * prm_ref[0:1, :] + prm_ref[1:2, :],
        0.0)
    o_ref[...] = jnp.max(h2.reshape(QBLK, NS, CF), axis=1)


def _mlp1(G, C, prm1):
    return pl.pallas_call(
        _mm1_body,
        grid=(GRID,),
        in_specs=[
            pl.BlockSpec((BLK, CF), lambda g: (g, 0)),
            pl.BlockSpec((QBLK, CF), lambda g: (g, 0)),
            pl.BlockSpec((8, CF), lambda g: (0, 0)),
        ],
        out_specs=[
            pl.BlockSpec((BLK, CF), lambda g: (g, 0)),
            pl.BlockSpec((8, CF), lambda g: (0, 0)),
        ],
        out_shape=[
            jax.ShapeDtypeStruct((GROWS, CF), jnp.bfloat16),
            jax.ShapeDtypeStruct((8, CF), jnp.float32),
        ],
    )(G, C, prm1)


def _mlp2(y1, W2, prm2):
    return pl.pallas_call(
        _mm2_body,
        grid=(GRID,),
        in_specs=[
            pl.BlockSpec((BLK, CF), lambda g: (g, 0)),
            pl.BlockSpec((CF, CF), lambda g: (0, 0)),
            pl.BlockSpec((8, CF), lambda g: (0, 0)),
        ],
        out_specs=[
            pl.BlockSpec((BLK, CF), lambda g: (g, 0)),
            pl.BlockSpec((8, CF), lambda g: (0, 0)),
        ],
        out_shape=[
            jax.ShapeDtypeStruct((GROWS, CF), jnp.bfloat16),
            jax.ShapeDtypeStruct((8, CF), jnp.float32),
        ],
    )(y1, W2, prm2)


def _outk(y2, prm3):
    return pl.pallas_call(
        _out_body,
        grid=(GRID,),
        in_specs=[
            pl.BlockSpec((BLK, CF), lambda g: (g, 0)),
            pl.BlockSpec((8, CF), lambda g: (0, 0)),
        ],
        out_specs=pl.BlockSpec((QBLK, CF), lambda g: (g, 0)),
        out_shape=jax.ShapeDtypeStruct((NPAD, CF), jnp.float32),
    )(y2, prm3)


def kernel(p, x, W1, b1, g1, beta1, W2, b2, g2, beta2, b):
    f32 = jnp.float32
    padv = 1e6 + jnp.arange(N, NPAD, dtype=f32)
    Ppad = jnp.concatenate([p, jnp.stack([padv, padv, padv], axis=1)])
    jj = jnp.arange(CB)
    PK = jnp.where(jj[:, None] // 4 == jnp.arange(WBLK)[None, :],
                   (2.0 ** (jj % 4))[:, None], 0.0).astype(jnp.bfloat16)
    BW = jnp.where(jj[:, None] // 64 == jnp.arange(128)[None, :],
                   1.0, 0.0).astype(jnp.bfloat16)
    T = jnp.concatenate([p, x, jnp.zeros((N, CIN - 3 - CF), f32)], axis=1)
    T = jnp.concatenate(
        [T, T[N - 1:N], jnp.zeros((NPAD - N - 1, CIN), f32)], axis=0)
    W1pad = jnp.concatenate([W1, jnp.zeros((CIN - 3 - CF, CF), f32)])
    PT = Ppad.T
    H = NPAD // 2
    W0, WC0 = _maskk(Ppad[:H], PT, PK, BW)
    idx0 = _bq_half(W0, WC0)
    W1, WC1 = _maskk(Ppad[H:], PT, PK, BW)
    idx1 = _bq_half(W1, WC1)
    idx_full = jnp.concatenate([idx0, idx1])
    idx_r = idx_full.reshape(NW, GNC, GCH)
    Pq = jnp.concatenate([p, jnp.zeros((NPAD - N, 3), f32)])
    U, C = _projk(T, Pq, W1pad)
    G = _gather(U, idx_r)

    prm1 = jnp.zeros((8, CF), f32).at[0].set(b1)
    y1, st1 = _mlp1(G, C, prm1)

    cnt = f32(VROWS)
    mu1 = st1[0] / cnt
    var1 = st1[1] / cnt - mu1 * mu1
    sc1 = g1 / jnp.sqrt(var1 + 1e-5)
    sh1 = beta1 - mu1 * sc1
    prm2 = jnp.zeros((8, CF), f32).at[0].set(sc1).at[1].set(sh1).at[2].set(b2)
    y2, st2 = _mlp2(y1, W2, prm2)

    mu2 = st2[0] / cnt
    var2 = st2[1] / cnt - mu2 * mu2
    sc2 = g2 / jnp.sqrt(var2 + 1e-5)
    sh2 = beta2 - mu2 * sc2
    prm3 = jnp.zeros((8, CF), f32).at[0].set(sc2).at[1].set(sh2)
    out = _outk(y2, prm3)
    return out[:N]


# speedup vs baseline: 1.3831x; 1.0554x over previous
import functools

import jax
import jax.numpy as jnp
from jax import lax
from jax.experimental import pallas as pl
from jax.experimental.pallas import tpu as pltpu
from jax.experimental.pallas import tpu_sc as plsc

N = 10000
NS = 16
R2 = 0.01
CF = 64
NW = 32
QPW = 320
NPAD = NW * QPW
NCH = NPAD // 16
CIN = 80
GROWS = NPAD * NS
VROWS = N * NS
BLK = 2048
QBLK = BLK // NS
GRID = GROWS // BLK
GCH = 128
GNC = GROWS // (NW * GCH)

RB = 512
CB = 1024

_SC_PARAMS = pltpu.CompilerParams(needs_layout_passes=False)


NWORD = NPAD // 4
NWIN = NPAD // 64
WBLK = NWORD // (NPAD // CB)
WCBLK = NWIN // (NPAD // CB)


def _sq_mask(pq_ref, pt_ref):
    pb = pq_ref[...]
    pt = pt_ref[...]
    pr2 = pb * pb
    pn_r = pr2[:, 0:1] + pr2[:, 1:2] + pr2[:, 2:3]
    pc2 = pt * pt
    pn_c = pc2[0:1, :] + pc2[1:2, :] + pc2[2:3, :]
    dot = jnp.dot(pb.astype(jnp.bfloat16), pt.astype(jnp.bfloat16),
                  preferred_element_type=jnp.float32)
    sq = (pn_r + pn_c) - 2.0 * dot
    return (sq <= R2).astype(jnp.bfloat16)


NWC = 1280


def _mask_body(pq_ref, pt_ref, pk_ref, bw_ref, w_ref, wc_ref):
    mb = _sq_mask(pq_ref, pt_ref)
    w_ref[...] = jnp.dot(mb, pk_ref[...], preferred_element_type=jnp.float32)
    wc_ref[...] = jnp.dot(mb, bw_ref[...], preferred_element_type=jnp.float32)


def _maskk(Ppad, PT, PK, BW):
    rows = Ppad.shape[0]
    return pl.pallas_call(
        _mask_body,
        grid=(rows // RB, NPAD // CB),
        in_specs=[
            pl.BlockSpec((RB, 3), lambda r, c: (r, 0)),
            pl.BlockSpec((3, CB), lambda r, c: (0, c)),
            pl.BlockSpec((CB, WBLK), lambda r, c: (0, 0)),
            pl.BlockSpec((CB, 128), lambda r, c: (0, 0)),
        ],
        out_specs=[
            pl.BlockSpec((RB, WBLK), lambda r, c: (r, c)),
            pl.BlockSpec((RB, 128), lambda r, c: (r, c)),
        ],
        out_shape=[
            jax.ShapeDtypeStruct((rows, NWORD), jnp.float32),
            jax.ShapeDtypeStruct((rows, NWC), jnp.float32),
        ],
    )(Ppad, PT, PK, BW)


_REAL_GRPS = list(range(0, 80, 8))


def _make_bq_body(qpw):
  def _bq_body(w_h, wc_h, out_h, row0, row1, cnt0, cnt1, wlist, cbase, buf,
               stage, sem0, sem1):
    wid = lax.axis_index("s") * 2 + lax.axis_index("c")
    lanes = lax.iota(jnp.int32, 16)
    base = wid * qpw
    wlist[...] = jnp.zeros((16,), jnp.int32)
    cbase[...] = jnp.zeros((16,), jnp.int32)
    pltpu.make_async_copy(w_h.at[base], row0, sem0).start()
    pltpu.make_async_copy(wc_h.at[base], cnt0, sem0).start()

    def process(q, rowv, cntv):
        run = jnp.zeros((16,), jnp.int32)
        nf = jnp.zeros((16,), jnp.int32)
        for g in _REAL_GRPS:
            wbase = (g // 8) * 16
            cwi = cntv[pl.ds(g * 16, 16)].astype(jnp.int32)
            cums = plsc.cumsum(cwi)
            cume = run + cums - cwi
            flag = (cwi > 0) & (cume < NS)
            fpos = nf + plsc.cumsum(flag.astype(jnp.int32)) - 1
            wm = flag & (fpos < 16)
            plsc.store_scatter(wlist, [fpos], wbase + lanes, mask=wm)
            plsc.store_scatter(cbase, [fpos], cume, mask=wm)
            nf = nf + plsc.all_reduce_population_count(flag)
            run = run + cums[jnp.zeros((16,), jnp.int32) + 15]
        wl = wlist[...]
        cb = cbase[...]

        for k in range(16):
            kv = nf > k
            w = wl[k]
            wi = rowv[pl.ds(w * 16, 16)].astype(jnp.int32)
            f0 = wi & 1
            f1 = (wi >> 1) & 1
            f2 = (wi >> 2) & 1
            f3 = (wi >> 3) & 1
            cwl = f0 + f1 + f2 + f3
            pexc = plsc.cumsum(cwl) - cwl
            cnd = w * 64 + 4 * lanes
            pos0 = cb[k] + pexc
            plsc.store_scatter(buf, [pos0], cnd,
                               mask=kv & (f0 > 0) & (pos0 < NS))
            pos1 = pos0 + f0
            plsc.store_scatter(buf, [pos1], cnd + 1,
                               mask=kv & (f1 > 0) & (pos1 < NS))
            pos2 = pos1 + f1
            plsc.store_scatter(buf, [pos2], cnd + 2,
                               mask=kv & (f2 > 0) & (pos2 < NS))
            pos3 = pos2 + f2
            plsc.store_scatter(buf, [pos3], cnd + 3,
                               mask=kv & (f3 > 0) & (pos3 < NS))

        vals = buf[...]
        first = jnp.where(run > 0, vals[jnp.zeros((16,), jnp.int32)], N)
        stage[q, :] = jnp.where(lanes < run, vals, first)

    def pair(t, carry):
        q0 = 2 * t
        q1 = 2 * t + 1
        pltpu.make_async_copy(w_h.at[base + q1], row1, sem1).start()
        pltpu.make_async_copy(wc_h.at[base + q1], cnt1, sem1).start()
        pltpu.make_async_copy(w_h.at[base + q0], row0, sem0).wait()
        pltpu.make_async_copy(wc_h.at[base + q0], cnt0, sem0).wait()
        process(q0, row0, cnt0)

        @pl.when(t < qpw // 2 - 1)
        def _():
            pltpu.make_async_copy(w_h.at[base + q1 + 1], row0, sem0).start()
            pltpu.make_async_copy(wc_h.at[base + q1 + 1], cnt0, sem0).start()

        pltpu.make_async_copy(w_h.at[base + q1], row1, sem1).wait()
        pltpu.make_async_copy(wc_h.at[base + q1], cnt1, sem1).wait()
        process(q1, row1, cnt1)
        return carry

    lax.fori_loop(0, qpw // 2, pair, jnp.int32(0))
    pltpu.sync_copy(stage, out_h.at[pl.ds(base, qpw)])

  return _bq_body


def _make_bq(qpw):
    rows = qpw * NW
    return functools.partial(
        pl.kernel,
        compiler_params=_SC_PARAMS,
        out_type=jax.ShapeDtypeStruct((rows, NS), jnp.int32),
        mesh=plsc.VectorSubcoreMesh(core_axis_name="c", subcore_axis_name="s"),
        scratch_types=[
            pltpu.VMEM((NWORD,), jnp.float32),
            pltpu.VMEM((NWORD,), jnp.float32),
            pltpu.VMEM((NWC,), jnp.float32),
            pltpu.VMEM((NWC,), jnp.float32),
            pltpu.VMEM((16,), jnp.int32),
            pltpu.VMEM((16,), jnp.int32),
            pltpu.VMEM((NS,), jnp.int32),
            pltpu.VMEM((qpw, NS), jnp.int32),
            pltpu.SemaphoreType.DMA,
            pltpu.SemaphoreType.DMA,
        ],
    )(_make_bq_body(qpw))


_bq_half = _make_bq(QPW // 4)


def _gather_body(tab_h, idx_h, out_h, idxv, rows0, rows1, sem0, sem1):
    wid = lax.axis_index("s") * 2 + lax.axis_index("c")
    pltpu.sync_copy(idx_h.at[wid], idxv)
    obase = wid * (GNC * GCH)
    pltpu.make_async_copy(tab_h.at[idxv.at[0]], rows0, sem0).start()

    def step(t, carry):
        j0 = 2 * t
        j1 = 2 * t + 1
        pltpu.make_async_copy(tab_h.at[idxv.at[j1]], rows1, sem1).start()
        pltpu.make_async_copy(tab_h.at[idxv.at[j0]], rows0, sem0).wait()
        pltpu.sync_copy(rows0, out_h.at[pl.ds(obase + j0 * GCH, GCH)])

        @pl.when(t < GNC // 2 - 1)
        def _():
            pltpu.make_async_copy(tab_h.at[idxv.at[j1 + 1]], rows0, sem0).start()

        pltpu.make_async_copy(tab_h.at[idxv.at[j1]], rows1, sem1).wait()
        pltpu.sync_copy(rows1, out_h.at[pl.ds(obase + j1 * GCH, GCH)])
        return carry

    lax.fori_loop(0, GNC // 2, step, jnp.int32(0))


_gather = functools.partial(
    pl.kernel,
    compiler_params=pltpu.CompilerParams(
        needs_layout_passes=False, use_tc_tiling_on_sc=False),
    out_type=jax.ShapeDtypeStruct((GROWS, CF), jnp.float32),
    mesh=plsc.VectorSubcoreMesh(core_axis_name="c", subcore_axis_name="s"),
    scratch_types=[
        pltpu.VMEM((GNC, GCH), jnp.int32),
        pltpu.VMEM((GCH, CF), jnp.float32),
        pltpu.VMEM((GCH, CF), jnp.float32),
        pltpu.SemaphoreType.DMA,
        pltpu.SemaphoreType.DMA,
    ],
)(_gather_body)


def _proj_body(t_ref, p_ref, w_ref, u_ref, c_ref):
    w = w_ref[...]
    u_ref[...] = jnp.dot(t_ref[...], w, preferred_element_type=jnp.float32)
    c_ref[...] = jnp.dot(p_ref[...], w[0:3, :], preferred_element_type=jnp.float32)


def _projk(T, Ppad, W1pad):
    return pl.pallas_call(
        _proj_body,
        grid=(5,),
        in_specs=[
            pl.BlockSpec((NPAD // 5, CIN), lambda r: (r, 0)),
            pl.BlockSpec((NPAD // 5, 3), lambda r: (r, 0)),
            pl.BlockSpec((CIN, CF), lambda r: (0, 0)),
        ],
        out_specs=[
            pl.BlockSpec((NPAD // 5, CF), lambda r: (r, 0)),
            pl.BlockSpec((NPAD // 5, CF), lambda r: (r, 0)),
        ],
        out_shape=[
            jax.ShapeDtypeStruct((NPAD, CF), jnp.float32),
            jax.ShapeDtypeStruct((NPAD, CF), jnp.float32),
        ],
    )(T, Ppad, W1pad)


def _mm1_body(g_ref, c_ref, prm_ref, y_ref, s_ref):
    g = pl.program_id(0)
    hw = g_ref[...]
    corr = c_ref[...]
    y = (hw.reshape(QBLK, NS, CF) - corr[:, None, :]).reshape(BLK, CF)
    y = y + prm_ref[0:1, :]
    y_ref[...] = y.astype(jnp.bfloat16)
    rid = lax.broadcasted_iota(jnp.int32, (BLK, 1), 0) + g * BLK
    ym = jnp.where(rid < VROWS, y, 0.0)

    @pl.when(g == 0)
    def _():
        s_ref[...] = jnp.zeros_like(s_ref)

    s_ref[0:1, :] += jnp.sum(ym, axis=0, keepdims=True)
    s_ref[1:2, :] += jnp.sum(ym * ym, axis=0, keepdims=True)


def _mm2_body(y1_ref, w_ref, prm_ref, y_ref, s_ref):
    g = pl.program_id(0)
    h1 = jnp.maximum(
        y1_ref[...].astype(jnp.float32) * prm_ref[0:1, :] + prm_ref[1:2, :],
        0.0)
    y = jnp.dot(h1, w_ref[...], preferred_element_type=jnp.float32)
    y = y + prm_ref[2:3, :]
    y_ref[...] = y.astype(jnp.bfloat16)
    rid = lax.broadcasted_iota(jnp.int32, (BLK, 1), 0) + g * BLK
    ym = jnp.where(rid < VROWS, y, 0.0)

    @pl.when(g == 0)
    def _():
        s_ref[...] = jnp.zeros_like(s_ref)

    s_ref[0:1, :] += jnp.sum(ym, axis=0, keepdims=True)
    s_ref[1:2, :] += jnp.sum(ym * ym, axis=0, keepdims=True)


def _out_body(y2_ref, prm_ref, o_ref):
    h2 = jnp.maximum(
        y2_ref[...].astype(jnp.float32) * prm_ref[0:1, :] + prm_ref[1:2, :],
        0.0)
    o_ref[...] = jnp.max(h2.reshape(QBLK, NS, CF), axis=1)


def _mlp1(G, C, prm1):
    return pl.pallas_call(
        _mm1_body,
        grid=(GRID,),
        in_specs=[
            pl.BlockSpec((BLK, CF), lambda g: (g, 0)),
            pl.BlockSpec((QBLK, CF), lambda g: (g, 0)),
            pl.BlockSpec((8, CF), lambda g: (0, 0)),
        ],
        out_specs=[
            pl.BlockSpec((BLK, CF), lambda g: (g, 0)),
            pl.BlockSpec((8, CF), lambda g: (0, 0)),
        ],
        out_shape=[
            jax.ShapeDtypeStruct((GROWS, CF), jnp.bfloat16),
            jax.ShapeDtypeStruct((8, CF), jnp.float32),
        ],
    )(G, C, prm1)


def _mlp2(y1, W2, prm2):
    return pl.pallas_call(
        _mm2_body,
        grid=(GRID,),
        in_specs=[
            pl.BlockSpec((BLK, CF), lambda g: (g, 0)),
            pl.BlockSpec((CF, CF), lambda g: (0, 0)),
            pl.BlockSpec((8, CF), lambda g: (0, 0)),
        ],
        out_specs=[
            pl.BlockSpec((BLK, CF), lambda g: (g, 0)),
            pl.BlockSpec((8, CF), lambda g: (0, 0)),
        ],
        out_shape=[
            jax.ShapeDtypeStruct((GROWS, CF), jnp.bfloat16),
            jax.ShapeDtypeStruct((8, CF), jnp.float32),
        ],
    )(y1, W2, prm2)


def _outk(y2, prm3):
    return pl.pallas_call(
        _out_body,
        grid=(GRID,),
        in_specs=[
            pl.BlockSpec((BLK, CF), lambda g: (g, 0)),
            pl.BlockSpec((8, CF), lambda g: (0, 0)),
        ],
        out_specs=pl.BlockSpec((QBLK, CF), lambda g: (g, 0)),
        out_shape=jax.ShapeDtypeStruct((NPAD, CF), jnp.float32),
    )(y2, prm3)


def kernel(p, x, W1, b1, g1, beta1, W2, b2, g2, beta2, b):
    f32 = jnp.float32
    padv = 1e6 + jnp.arange(N, NPAD, dtype=f32)
    Ppad = jnp.concatenate([p, jnp.stack([padv, padv, padv], axis=1)])
    jj = jnp.arange(CB)
    PK = jnp.where(jj[:, None] // 4 == jnp.arange(WBLK)[None, :],
                   (2.0 ** (jj % 4))[:, None], 0.0).astype(jnp.bfloat16)
    BW = jnp.where(jj[:, None] // 64 == jnp.arange(128)[None, :],
                   1.0, 0.0).astype(jnp.bfloat16)
    T = jnp.concatenate([p, x, jnp.zeros((N, CIN - 3 - CF), f32)], axis=1)
    T = jnp.concatenate(
        [T, T[N - 1:N], jnp.zeros((NPAD - N - 1, CIN), f32)], axis=0)
    W1pad = jnp.concatenate([W1, jnp.zeros((CIN - 3 - CF, CF), f32)])
    PT = Ppad.T
    H = NPAD // 4
    parts = []
    for h in range(4):
        Wh, WCh = _maskk(Ppad[h * H:(h + 1) * H], PT, PK, BW)
        parts.append(_bq_half(Wh, WCh))
    idx_full = jnp.concatenate(parts)
    idx_r = idx_full.reshape(NW, GNC, GCH)
    Pq = jnp.concatenate([p, jnp.zeros((NPAD - N, 3), f32)])
    U, C = _projk(T, Pq, W1pad)
    G = _gather(U, idx_r)

    prm1 = jnp.zeros((8, CF), f32).at[0].set(b1)
    y1, st1 = _mlp1(G, C, prm1)

    cnt = f32(VROWS)
    mu1 = st1[0] / cnt
    var1 = st1[1] / cnt - mu1 * mu1
    sc1 = g1 / jnp.sqrt(var1 + 1e-5)
    sh1 = beta1 - mu1 * sc1
    prm2 = jnp.zeros((8, CF), f32).at[0].set(sc1).at[1].set(sh1).at[2].set(b2)
    y2, st2 = _mlp2(y1, W2, prm2)

    mu2 = st2[0] / cnt
    var2 = st2[1] / cnt - mu2 * mu2
    sc2 = g2 / jnp.sqrt(var2 + 1e-5)
    sh2 = beta2 - mu2 * sc2
    prm3 = jnp.zeros((8, CF), f32).at[0].set(sc2).at[1].set(sh2)
    out = _outk(y2, prm3)
    return out[:N]
